# Initial kernel scaffold; baseline (speedup 1.0000x reference)
import jax
import jax.numpy as jnp
from jax.experimental import pallas as pl


def kernel(hidden_states, gate_w, w_in, w_v, w_out):
    raise NotImplementedError("write your pallas kernel here")


# speedup vs baseline: 1.5351x; 1.5351x over previous
import functools

import jax
import jax.numpy as jnp
from jax.experimental import pallas as pl


def _router_body(x_ref, gw_ref, logits_ref, coef_ref):
    x = x_ref[...]
    gw = gw_ref[...]
    logits = jnp.dot(x, gw, preferred_element_type=jnp.float32)
    S, E = logits.shape
    iota = jax.lax.broadcasted_iota(jnp.int32, (S, E), 1)
    m1 = jnp.max(logits, axis=1, keepdims=True)
    idx1 = jnp.min(jnp.where(logits == m1, iota, E), axis=1, keepdims=True)
    oh1 = iota == idx1
    masked = jnp.where(oh1, -jnp.inf, logits)
    m2 = jnp.max(masked, axis=1, keepdims=True)
    idx2 = jnp.min(jnp.where(masked == m2, iota, E), axis=1, keepdims=True)
    oh2 = iota == idx2
    w1 = 1.0 / (1.0 + jnp.exp(m2 - m1))
    w2 = 1.0 - w1
    logits_ref[...] = logits
    coef_ref[...] = jnp.where(oh1, w1, 0.0) + jnp.where(oh2, w2, 0.0)


def _moe_body(x_ref, coef_ref, wi_ref, wv_ref, wo_ref, out_ref):
    e = pl.program_id(0)
    j = pl.program_id(1)

    @pl.when((e == 0) & (j == 0))
    def _init():
        out_ref[...] = jnp.zeros_like(out_ref)

    xb = x_ref[...].astype(jnp.bfloat16)
    a = jnp.dot(xb, wi_ref[0].astype(jnp.bfloat16),
                preferred_element_type=jnp.float32)
    b = jnp.dot(xb, wv_ref[0].astype(jnp.bfloat16),
                preferred_element_type=jnp.float32)
    h = (jax.nn.gelu(a) * b).astype(jnp.bfloat16)
    y = jnp.dot(h, wo_ref[0].astype(jnp.bfloat16),
                preferred_element_type=jnp.float32)
    coef = coef_ref[...]
    E = coef.shape[1]
    iota = jax.lax.broadcasted_iota(jnp.int32, coef.shape, 1)
    ccol = jnp.sum(jnp.where(iota == e, coef, 0.0), axis=1, keepdims=True)
    out_ref[...] += ccol * y


@functools.partial(jax.jit, static_argnums=())
def kernel(hidden_states, gate_w, w_in, w_v, w_out):
    B, S, D = hidden_states.shape
    E = gate_w.shape[1]
    F = w_in.shape[2]
    x = hidden_states.reshape(S, D)

    logits, coef = pl.pallas_call(
        _router_body,
        out_shape=(
            jax.ShapeDtypeStruct((S, E), jnp.float32),
            jax.ShapeDtypeStruct((S, E), jnp.float32),
        ),
    )(x, gate_w)

    JF = 4
    Fb = F // JF
    out = pl.pallas_call(
        _moe_body,
        grid=(E, JF),
        in_specs=[
            pl.BlockSpec((S, D), lambda e, j: (0, 0)),
            pl.BlockSpec((S, E), lambda e, j: (0, 0)),
            pl.BlockSpec((1, D, Fb), lambda e, j: (e, 0, j)),
            pl.BlockSpec((1, D, Fb), lambda e, j: (e, 0, j)),
            pl.BlockSpec((1, Fb, D), lambda e, j: (e, j, 0)),
        ],
        out_specs=pl.BlockSpec((S, D), lambda e, j: (0, 0)),
        out_shape=jax.ShapeDtypeStruct((S, D), jnp.float32),
    )(x, coef, w_in, w_v, w_out)

    return out.reshape(B, S, D), logits.reshape(B, S, E)


# speedup vs baseline: 1.8081x; 1.1778x over previous
import functools

import jax
import jax.numpy as jnp
from jax import lax
from jax.experimental import pallas as pl
from jax.experimental.pallas import tpu as pltpu
from jax.experimental.pallas import tpu_sc as plsc

S, D, F, E = 2048, 768, 3072, 8
T = 512
NBLK = 16
NSLOT = NBLK * T
JF = 2
Fb = F // JF
SB = 8
RB = S // SB
NC, NS = 2, 16
NW = NC * NS
TOK_W = S // NW


def _router_body(x_ref, gw_ref, logits_ref, idx1_ref, idx2_ref,
                 w1_ref, w2_ref, posin_ref, counts_ref, cnt_acc):
    i = pl.program_id(0)

    @pl.when(i == 0)
    def _init():
        cnt_acc[...] = jnp.zeros_like(cnt_acc)

    logits = jnp.dot(x_ref[...], gw_ref[...],
                     preferred_element_type=jnp.float32)
    iota = lax.broadcasted_iota(jnp.int32, (RB, E), 1)
    m1 = jnp.max(logits, axis=1, keepdims=True)
    idx1 = jnp.min(jnp.where(logits == m1, iota, E), axis=1, keepdims=True)
    oh1 = iota == idx1
    masked = jnp.where(oh1, -jnp.inf, logits)
    m2 = jnp.max(masked, axis=1, keepdims=True)
    idx2 = jnp.min(jnp.where(masked == m2, iota, E), axis=1, keepdims=True)
    oh2 = iota == idx2
    w1 = 1.0 / (1.0 + jnp.exp(m2 - m1))

    mask = (oh1 | oh2).astype(jnp.float32)
    r = lax.broadcasted_iota(jnp.int32, (RB, RB), 0)
    c = lax.broadcasted_iota(jnp.int32, (RB, RB), 1)
    tril = (r > c).astype(jnp.bfloat16)
    posin = jnp.dot(tril, mask.astype(jnp.bfloat16),
                    preferred_element_type=jnp.float32) + cnt_acc[...]

    logits_ref[...] = logits
    idx1_ref[...] = idx1
    idx2_ref[...] = idx2
    w1_ref[...] = w1
    w2_ref[...] = 1.0 - w1
    posin_ref[...] = posin
    cnt_acc[...] += jnp.sum(mask, axis=0, keepdims=True)

    @pl.when(i == SB - 1)
    def _fin():
        counts_ref[...] = cnt_acc[...]


def _a2_body(counts_ref, idx1_ref, idx2_ref, posin_ref,
             slot1_ref, slot2_ref, be_ref, nt_ref):
    counts = counts_ref[...]
    nb = jnp.ceil(counts / T)
    lane8 = lax.broadcasted_iota(jnp.int32, (1, E), 1)
    baseb = jnp.zeros_like(nb)
    for e in range(E - 1):
        baseb += jnp.where(lane8 > e, nb[:, e:e + 1], 0.0)
    cum_incl = baseb + nb
    ntot = jnp.sum(nb, axis=1, keepdims=True)

    base_slots = baseb * T
    slotmat = base_slots + posin_ref[...]
    iota = lax.broadcasted_iota(jnp.int32, (S, E), 1)
    oh1 = iota == idx1_ref[...]
    oh2 = iota == idx2_ref[...]
    slot1_ref[...] = jnp.sum(jnp.where(oh1, slotmat, 0.0),
                             axis=1, keepdims=True).astype(jnp.int32)
    slot2_ref[...] = jnp.sum(jnp.where(oh2, slotmat, 0.0),
                             axis=1, keepdims=True).astype(jnp.int32)

    ib = lax.broadcasted_iota(jnp.int32, (1, NBLK), 1).astype(jnp.float32)
    raw = jnp.zeros((1, NBLK), jnp.float32)
    el = jnp.zeros((1, 1), jnp.float32)
    for e in range(E):
        raw += (cum_incl[:, e:e + 1] <= ib).astype(jnp.float32)
        el += (cum_incl[:, e:e + 1] <= ntot - 1.0).astype(jnp.float32)
    be_ref[...] = jnp.minimum(raw, el).astype(jnp.int32)
    nt_ref[...] = ntot.astype(jnp.int32)


def _dispatch_body(x_hbm, s1_hbm, s2_hbm, w1_hbm, w2_hbm,
                   xs_hbm, ws_hbm, xb, i1, i2, wv1, wv2, sem):
    wid = lax.axis_index("s") * NC + lax.axis_index("c")
    base = wid * TOK_W
    pltpu.sync_copy(s1_hbm.at[pl.ds(base, TOK_W)], i1)
    pltpu.sync_copy(s2_hbm.at[pl.ds(base, TOK_W)], i2)
    pltpu.sync_copy(w1_hbm.at[pl.ds(base, TOK_W)], wv1)
    pltpu.sync_copy(w2_hbm.at[pl.ds(base, TOK_W)], wv2)
    pltpu.sync_copy(x_hbm.at[pl.ds(base, TOK_W)], xb)
    c1 = pltpu.async_copy(xb, xs_hbm.at[i1], sem)
    c2 = pltpu.async_copy(xb, xs_hbm.at[i2], sem)
    c3 = pltpu.async_copy(wv1, ws_hbm.at[i1], sem)
    c4 = pltpu.async_copy(wv2, ws_hbm.at[i2], sem)
    c1.wait()
    c2.wait()
    c3.wait()
    c4.wait()


def _mlp_body(be_ref, nt_ref, xs_ref, ws_ref, wi_ref, wv_ref, wo_ref, ys_ref):
    i = pl.program_id(0)
    j = pl.program_id(1)

    @pl.when(i < nt_ref[0])
    def _compute():
        xb = xs_ref[...].astype(jnp.bfloat16)
        a = jnp.dot(xb, wi_ref[0].astype(jnp.bfloat16),
                    preferred_element_type=jnp.float32)
        b = jnp.dot(xb, wv_ref[0].astype(jnp.bfloat16),
                    preferred_element_type=jnp.float32)
        h = (jax.nn.gelu(a) * b).astype(jnp.bfloat16)
        y = jnp.dot(h, wo_ref[0].astype(jnp.bfloat16),
                    preferred_element_type=jnp.float32)
        wy = ws_ref[...] * y

        @pl.when(j == 0)
        def _set():
            ys_ref[...] = wy

        @pl.when(j != 0)
        def _acc():
            ys_ref[...] += wy


def _combine_body(ys_hbm, s1_hbm, s2_hbm, out_hbm, i1, i2, b1, b2, sem):
    wid = lax.axis_index("s") * NC + lax.axis_index("c")
    base = wid * TOK_W
    pltpu.sync_copy(s1_hbm.at[pl.ds(base, TOK_W)], i1)
    pltpu.sync_copy(s2_hbm.at[pl.ds(base, TOK_W)], i2)
    c1 = pltpu.async_copy(ys_hbm.at[i1], b1, sem)
    c2 = pltpu.async_copy(ys_hbm.at[i2], b2, sem)
    c1.wait()
    c2.wait()

    def row(t, _):
        def col(c, _):
            b1[t, pl.ds(c * 16, 16)] += b2[t, pl.ds(c * 16, 16)]
            return 0
        return lax.fori_loop(0, D // 16, col, 0, unroll=8)

    lax.fori_loop(0, TOK_W, row, 0)
    pltpu.sync_copy(b1, out_hbm.at[pl.ds(base, TOK_W)])


def kernel(hidden_states, gate_w, w_in, w_v, w_out):
    B = hidden_states.shape[0]
    x = hidden_states.reshape(S, D)

    logits, idx1, idx2, w1, w2, posin, counts = pl.pallas_call(
        _router_body,
        grid=(SB,),
        in_specs=[
            pl.BlockSpec((RB, D), lambda i: (i, 0)),
            pl.BlockSpec((D, E), lambda i: (0, 0)),
        ],
        out_specs=[
            pl.BlockSpec((RB, E), lambda i: (i, 0)),
            pl.BlockSpec((RB, 1), lambda i: (i, 0)),
            pl.BlockSpec((RB, 1), lambda i: (i, 0)),
            pl.BlockSpec((RB, 1), lambda i: (i, 0)),
            pl.BlockSpec((RB, 1), lambda i: (i, 0)),
            pl.BlockSpec((RB, E), lambda i: (i, 0)),
            pl.BlockSpec((1, E), lambda i: (0, 0)),
        ],
        out_shape=(
            jax.ShapeDtypeStruct((S, E), jnp.float32),
            jax.ShapeDtypeStruct((S, 1), jnp.int32),
            jax.ShapeDtypeStruct((S, 1), jnp.int32),
            jax.ShapeDtypeStruct((S, 1), jnp.float32),
            jax.ShapeDtypeStruct((S, 1), jnp.float32),
            jax.ShapeDtypeStruct((S, E), jnp.float32),
            jax.ShapeDtypeStruct((1, E), jnp.float32),
        ),
        scratch_shapes=[pltpu.VMEM((1, E), jnp.float32)],
    )(x, gate_w)

    slot1, slot2, be, nt = pl.pallas_call(
        _a2_body,
        out_shape=(
            jax.ShapeDtypeStruct((S, 1), jnp.int32),
            jax.ShapeDtypeStruct((S, 1), jnp.int32),
            jax.ShapeDtypeStruct((1, NBLK), jnp.int32),
            jax.ShapeDtypeStruct((1, 1), jnp.int32),
        ),
    )(counts, idx1, idx2, posin)

    s1 = slot1.reshape(S)
    s2 = slot2.reshape(S)
    mesh = plsc.VectorSubcoreMesh(core_axis_name="c", subcore_axis_name="s")
    xs, ws = pl.kernel(
        _dispatch_body,
        out_type=(
            jax.ShapeDtypeStruct((NSLOT, D), jnp.float32),
            jax.ShapeDtypeStruct((NSLOT,), jnp.float32),
        ),
        mesh=mesh,
        scratch_types=[
            pltpu.VMEM((TOK_W, D), jnp.float32),
            pltpu.VMEM((TOK_W,), jnp.int32),
            pltpu.VMEM((TOK_W,), jnp.int32),
            pltpu.VMEM((TOK_W,), jnp.float32),
            pltpu.VMEM((TOK_W,), jnp.float32),
            pltpu.SemaphoreType.DMA,
        ],
    )(x, s1, s2, w1.reshape(S), w2.reshape(S))

    ys = pl.pallas_call(
        _mlp_body,
        grid_spec=pltpu.PrefetchScalarGridSpec(
            num_scalar_prefetch=2,
            grid=(NBLK, JF),
            in_specs=[
                pl.BlockSpec(
                    (T, D), lambda i, j, be, nt: (jnp.minimum(i, nt[0] - 1), 0)),
                pl.BlockSpec(
                    (T, 1), lambda i, j, be, nt: (jnp.minimum(i, nt[0] - 1), 0)),
                pl.BlockSpec(
                    (1, D, Fb),
                    lambda i, j, be, nt: (be[i], 0,
                                          jnp.where(i < nt[0], j, JF - 1))),
                pl.BlockSpec(
                    (1, D, Fb),
                    lambda i, j, be, nt: (be[i], 0,
                                          jnp.where(i < nt[0], j, JF - 1))),
                pl.BlockSpec(
                    (1, Fb, D),
                    lambda i, j, be, nt: (be[i],
                                          jnp.where(i < nt[0], j, JF - 1), 0)),
            ],
            out_specs=pl.BlockSpec(
                (T, D), lambda i, j, be, nt: (jnp.minimum(i, nt[0] - 1), 0)),
        ),
        out_shape=jax.ShapeDtypeStruct((NSLOT, D), jnp.float32),
    )(be.reshape(NBLK), nt.reshape(1), xs, ws.reshape(NSLOT, 1),
      w_in, w_v, w_out)

    out = pl.kernel(
        _combine_body,
        out_type=jax.ShapeDtypeStruct((S, D), jnp.float32),
        mesh=mesh,
        scratch_types=[
            pltpu.VMEM((TOK_W,), jnp.int32),
            pltpu.VMEM((TOK_W,), jnp.int32),
            pltpu.VMEM((TOK_W, D), jnp.float32),
            pltpu.VMEM((TOK_W, D), jnp.float32),
            pltpu.SemaphoreType.DMA,
        ],
    )(ys, s1, s2)

    return out.reshape(B, S, D), logits.reshape(B, S, E)


# speedup vs baseline: 1.9936x; 1.1026x over previous
import jax
import jax.numpy as jnp
from jax import lax
from jax.experimental import pallas as pl
from jax.experimental.pallas import tpu as pltpu
from jax.experimental.pallas import tpu_sc as plsc

S, D, F, E = 2048, 768, 3072, 8
DP = D // 2
T = 512
NBLK = 16
NSLOT = NBLK * T
JF = 2
Fb = F // JF
SB = 8
RB = S // SB
NC, NS = 2, 16
NW = NC * NS
TOK_W = S // NW


def _pack_bf16_pairs(x32):
    xu = lax.bitcast_convert_type(x32.astype(jnp.bfloat16), jnp.uint16)
    lo = xu[:, :DP].astype(jnp.uint32)
    hi = xu[:, DP:].astype(jnp.uint32)
    return lax.bitcast_convert_type(lo | (hi << 16), jnp.int32)


def _unpack_bf16_pairs(xi):
    xu = lax.bitcast_convert_type(xi, jnp.uint32)
    lo = (xu & 0xFFFF).astype(jnp.uint16)
    hi = (xu >> 16).astype(jnp.uint16)
    return jnp.concatenate(
        [lax.bitcast_convert_type(lo, jnp.bfloat16),
         lax.bitcast_convert_type(hi, jnp.bfloat16)], axis=1)


def _router_body(x_ref, gw_ref, logits_ref, w1_ref, w2_ref, xpk_ref,
                 slot1_ref, slot2_ref, be_ref, nt_ref,
                 cnt_acc, posin_s, i1_s, i2_s):
    i = pl.program_id(0)

    @pl.when(i == 0)
    def _init():
        cnt_acc[...] = jnp.zeros_like(cnt_acc)

    x = x_ref[...]
    logits = jnp.dot(x, gw_ref[...],
                     preferred_element_type=jnp.float32)
    iota = lax.broadcasted_iota(jnp.int32, (RB, E), 1)
    m1 = jnp.max(logits, axis=1, keepdims=True)
    idx1 = jnp.min(jnp.where(logits == m1, iota, E), axis=1, keepdims=True)
    oh1 = iota == idx1
    masked = jnp.where(oh1, -jnp.inf, logits)
    m2 = jnp.max(masked, axis=1, keepdims=True)
    idx2 = jnp.min(jnp.where(masked == m2, iota, E), axis=1, keepdims=True)
    oh2 = iota == idx2
    w1 = 1.0 / (1.0 + jnp.exp(m2 - m1))

    mask = (oh1 | oh2).astype(jnp.float32)
    r = lax.broadcasted_iota(jnp.int32, (RB, RB), 0)
    c = lax.broadcasted_iota(jnp.int32, (RB, RB), 1)
    tril = (r > c).astype(jnp.bfloat16)
    posin = jnp.dot(tril, mask.astype(jnp.bfloat16),
                    preferred_element_type=jnp.float32) + cnt_acc[...]

    logits_ref[...] = logits
    w1_ref[...] = w1
    w2_ref[...] = 1.0 - w1
    xpk_ref[...] = _pack_bf16_pairs(x)
    posin_s[pl.ds(i * RB, RB), :] = posin
    i1_s[pl.ds(i * RB, RB), :] = idx1
    i2_s[pl.ds(i * RB, RB), :] = idx2
    cnt_acc[...] += jnp.sum(mask, axis=0, keepdims=True)

    @pl.when(i == SB - 1)
    def _fin():
        counts = cnt_acc[...]
        nb = jnp.ceil(counts / T)
        lane8 = lax.broadcasted_iota(jnp.int32, (1, E), 1)
        baseb = jnp.zeros_like(nb)
        for e in range(E - 1):
            baseb += jnp.where(lane8 > e, nb[:, e:e + 1], 0.0)
        cum_incl = baseb + nb
        ntot = jnp.sum(nb, axis=1, keepdims=True)

        slotmat = baseb * T + posin_s[...]
        iota_se = lax.broadcasted_iota(jnp.int32, (S, E), 1)
        o1 = iota_se == i1_s[...]
        o2 = iota_se == i2_s[...]
        slot1_ref[...] = jnp.sum(jnp.where(o1, slotmat, 0.0),
                                 axis=1, keepdims=True).astype(jnp.int32)
        slot2_ref[...] = jnp.sum(jnp.where(o2, slotmat, 0.0),
                                 axis=1, keepdims=True).astype(jnp.int32)

        ib = lax.broadcasted_iota(jnp.int32, (1, NBLK), 1).astype(jnp.float32)
        raw = jnp.zeros((1, NBLK), jnp.float32)
        el = jnp.zeros((1, 1), jnp.float32)
        for e in range(E):
            raw += (cum_incl[:, e:e + 1] <= ib).astype(jnp.float32)
            el += (cum_incl[:, e:e + 1] <= ntot - 1.0).astype(jnp.float32)
        be_ref[...] = jnp.minimum(raw, el).astype(jnp.int32)
        nt_ref[...] = ntot.astype(jnp.int32)


def _dispatch_body(xpk_hbm, s1_hbm, s2_hbm, w1_hbm, w2_hbm,
                   xs_hbm, ws_hbm, xb, i1, i2, wv1, wv2, sem):
    wid = lax.axis_index("s") * NC + lax.axis_index("c")
    base = wid * TOK_W
    l1 = pltpu.async_copy(s1_hbm.at[pl.ds(base, TOK_W)], i1, sem)
    l2 = pltpu.async_copy(s2_hbm.at[pl.ds(base, TOK_W)], i2, sem)
    l3 = pltpu.async_copy(w1_hbm.at[pl.ds(base, TOK_W)], wv1, sem)
    l4 = pltpu.async_copy(w2_hbm.at[pl.ds(base, TOK_W)], wv2, sem)
    l5 = pltpu.async_copy(xpk_hbm.at[pl.ds(base, TOK_W)], xb, sem)
    l1.wait()
    l2.wait()
    l3.wait()
    l4.wait()
    l5.wait()
    c1 = pltpu.async_copy(xb, xs_hbm.at[i1], sem)
    c2 = pltpu.async_copy(xb, xs_hbm.at[i2], sem)
    c3 = pltpu.async_copy(wv1, ws_hbm.at[i1], sem)
    c4 = pltpu.async_copy(wv2, ws_hbm.at[i2], sem)
    c1.wait()
    c2.wait()
    c3.wait()
    c4.wait()


def _mlp_body(be_ref, nt_ref, xs_ref, ws_ref, wi_ref, wv_ref, wo_ref, ys_ref):
    i = pl.program_id(0)
    j = pl.program_id(1)

    @pl.when(i < nt_ref[0])
    def _compute():
        xb = _unpack_bf16_pairs(xs_ref[...])
        a = jnp.dot(xb, wi_ref[0].astype(jnp.bfloat16),
                    preferred_element_type=jnp.float32)
        b = jnp.dot(xb, wv_ref[0].astype(jnp.bfloat16),
                    preferred_element_type=jnp.float32)
        h = (jax.nn.gelu(a) * b).astype(jnp.bfloat16)
        y = jnp.dot(h, wo_ref[0].astype(jnp.bfloat16),
                    preferred_element_type=jnp.float32)
        wy = ws_ref[...] * y

        @pl.when(j == 0)
        def _set():
            ys_ref[...] = wy

        @pl.when(j != 0)
        def _acc():
            ys_ref[...] += wy


def _combine_body(ys_hbm, s1_hbm, s2_hbm, out_hbm, i1, i2, b1, b2, sem):
    wid = lax.axis_index("s") * NC + lax.axis_index("c")
    base = wid * TOK_W
    l1 = pltpu.async_copy(s1_hbm.at[pl.ds(base, TOK_W)], i1, sem)
    l2 = pltpu.async_copy(s2_hbm.at[pl.ds(base, TOK_W)], i2, sem)
    l1.wait()
    l2.wait()
    c1 = pltpu.async_copy(ys_hbm.at[i1], b1, sem)
    c2 = pltpu.async_copy(ys_hbm.at[i2], b2, sem)
    c1.wait()
    c2.wait()

    def row(t, _):
        def col(c, _):
            b1[t, pl.ds(c * 16, 16)] += b2[t, pl.ds(c * 16, 16)]
            return 0
        return lax.fori_loop(0, D // 16, col, 0, unroll=8)

    lax.fori_loop(0, TOK_W, row, 0)
    pltpu.sync_copy(b1, out_hbm.at[pl.ds(base, TOK_W)])


def kernel(hidden_states, gate_w, w_in, w_v, w_out):
    B = hidden_states.shape[0]
    x = hidden_states.reshape(S, D)

    logits, w1, w2, xpk, slot1, slot2, be, nt = pl.pallas_call(
        _router_body,
        grid=(SB,),
        in_specs=[
            pl.BlockSpec((RB, D), lambda i: (i, 0)),
            pl.BlockSpec((D, E), lambda i: (0, 0)),
        ],
        out_specs=[
            pl.BlockSpec((RB, E), lambda i: (i, 0)),
            pl.BlockSpec((RB, 1), lambda i: (i, 0)),
            pl.BlockSpec((RB, 1), lambda i: (i, 0)),
            pl.BlockSpec((RB, DP), lambda i: (i, 0)),
            pl.BlockSpec((S, 1), lambda i: (0, 0)),
            pl.BlockSpec((S, 1), lambda i: (0, 0)),
            pl.BlockSpec((1, NBLK), lambda i: (0, 0)),
            pl.BlockSpec((1, 1), lambda i: (0, 0)),
        ],
        out_shape=(
            jax.ShapeDtypeStruct((S, E), jnp.float32),
            jax.ShapeDtypeStruct((S, 1), jnp.float32),
            jax.ShapeDtypeStruct((S, 1), jnp.float32),
            jax.ShapeDtypeStruct((S, DP), jnp.int32),
            jax.ShapeDtypeStruct((S, 1), jnp.int32),
            jax.ShapeDtypeStruct((S, 1), jnp.int32),
            jax.ShapeDtypeStruct((1, NBLK), jnp.int32),
            jax.ShapeDtypeStruct((1, 1), jnp.int32),
        ),
        scratch_shapes=[
            pltpu.VMEM((1, E), jnp.float32),
            pltpu.VMEM((S, E), jnp.float32),
            pltpu.VMEM((S, 1), jnp.int32),
            pltpu.VMEM((S, 1), jnp.int32),
        ],
    )(x, gate_w)

    s1 = slot1.reshape(S)
    s2 = slot2.reshape(S)
    mesh = plsc.VectorSubcoreMesh(core_axis_name="c", subcore_axis_name="s")
    xs, ws = pl.kernel(
        _dispatch_body,
        out_type=(
            jax.ShapeDtypeStruct((NSLOT, DP), jnp.int32),
            jax.ShapeDtypeStruct((NSLOT,), jnp.float32),
        ),
        mesh=mesh,
        scratch_types=[
            pltpu.VMEM((TOK_W, DP), jnp.int32),
            pltpu.VMEM((TOK_W,), jnp.int32),
            pltpu.VMEM((TOK_W,), jnp.int32),
            pltpu.VMEM((TOK_W,), jnp.float32),
            pltpu.VMEM((TOK_W,), jnp.float32),
            pltpu.SemaphoreType.DMA,
        ],
    )(xpk, s1, s2, w1.reshape(S), w2.reshape(S))

    ys = pl.pallas_call(
        _mlp_body,
        grid_spec=pltpu.PrefetchScalarGridSpec(
            num_scalar_prefetch=2,
            grid=(NBLK, JF),
            in_specs=[
                pl.BlockSpec(
                    (T, DP), lambda i, j, be, nt: (jnp.minimum(i, nt[0] - 1), 0)),
                pl.BlockSpec(
                    (T, 1), lambda i, j, be, nt: (jnp.minimum(i, nt[0] - 1), 0)),
                pl.BlockSpec(
                    (1, D, Fb),
                    lambda i, j, be, nt: (be[i], 0,
                                          jnp.where(i < nt[0], j, JF - 1))),
                pl.BlockSpec(
                    (1, D, Fb),
                    lambda i, j, be, nt: (be[i], 0,
                                          jnp.where(i < nt[0], j, JF - 1))),
                pl.BlockSpec(
                    (1, Fb, D),
                    lambda i, j, be, nt: (be[i],
                                          jnp.where(i < nt[0], j, JF - 1), 0)),
            ],
            out_specs=pl.BlockSpec(
                (T, D), lambda i, j, be, nt: (jnp.minimum(i, nt[0] - 1), 0)),
        ),
        out_shape=jax.ShapeDtypeStruct((NSLOT, D), jnp.float32),
    )(be.reshape(NBLK), nt.reshape(1), xs, ws.reshape(NSLOT, 1),
      w_in, w_v, w_out)

    out = pl.kernel(
        _combine_body,
        out_type=jax.ShapeDtypeStruct((S, D), jnp.float32),
        mesh=mesh,
        scratch_types=[
            pltpu.VMEM((TOK_W,), jnp.int32),
            pltpu.VMEM((TOK_W,), jnp.int32),
            pltpu.VMEM((TOK_W, D), jnp.float32),
            pltpu.VMEM((TOK_W, D), jnp.float32),
            pltpu.SemaphoreType.DMA,
        ],
    )(ys, s1, s2)

    return out.reshape(B, S, D), logits.reshape(B, S, E)


# speedup vs baseline: 2.2659x; 1.1366x over previous
import jax
import jax.numpy as jnp
from jax import lax
from jax.experimental import pallas as pl
from jax.experimental.pallas import tpu as pltpu
from jax.experimental.pallas import tpu_sc as plsc

S, D, F, E = 2048, 768, 3072, 8
DP = D // 2
T = 512
NBLK = 16
NSLOT = NBLK * T
JF = 2
Fb = F // JF
SB = 8
RB = S // SB
NC, NS = 2, 16
NW = NC * NS
TOK_W = S // NW


def _pack_bf16_pairs(x32):
    xu = lax.bitcast_convert_type(x32.astype(jnp.bfloat16), jnp.uint16)
    lo = xu[:, :DP].astype(jnp.uint32)
    hi = xu[:, DP:].astype(jnp.uint32)
    return lax.bitcast_convert_type(lo | (hi << 16), jnp.int32)


def _unpack_bf16_pairs(xi):
    xu = lax.bitcast_convert_type(xi, jnp.uint32)
    lo = (xu & 0xFFFF).astype(jnp.uint16)
    hi = (xu >> 16).astype(jnp.uint16)
    return jnp.concatenate(
        [lax.bitcast_convert_type(lo, jnp.bfloat16),
         lax.bitcast_convert_type(hi, jnp.bfloat16)], axis=1)


def _router_body(x_ref, gw_ref, logits_ref, w1_ref, w2_ref, xpk_ref,
                 slot1_ref, slot2_ref, be_ref, nt_ref,
                 cnt_acc, posin_s, i1_s, i2_s):
    i = pl.program_id(0)

    @pl.when(i == 0)
    def _init():
        cnt_acc[...] = jnp.zeros_like(cnt_acc)

    x = x_ref[...]
    logits = jnp.dot(x, gw_ref[...],
                     preferred_element_type=jnp.float32)
    iota = lax.broadcasted_iota(jnp.int32, (RB, E), 1)
    m1 = jnp.max(logits, axis=1, keepdims=True)
    idx1 = jnp.min(jnp.where(logits == m1, iota, E), axis=1, keepdims=True)
    oh1 = iota == idx1
    masked = jnp.where(oh1, -jnp.inf, logits)
    m2 = jnp.max(masked, axis=1, keepdims=True)
    idx2 = jnp.min(jnp.where(masked == m2, iota, E), axis=1, keepdims=True)
    oh2 = iota == idx2
    w1 = 1.0 / (1.0 + jnp.exp(m2 - m1))

    mask = (oh1 | oh2).astype(jnp.float32)
    r = lax.broadcasted_iota(jnp.int32, (RB, RB), 0)
    c = lax.broadcasted_iota(jnp.int32, (RB, RB), 1)
    tril = (r > c).astype(jnp.bfloat16)
    posin = jnp.dot(tril, mask.astype(jnp.bfloat16),
                    preferred_element_type=jnp.float32) + cnt_acc[...]

    logits_ref[...] = logits
    w1_ref[...] = jnp.broadcast_to(w1, (RB, 16))
    w2_ref[...] = jnp.broadcast_to(1.0 - w1, (RB, 16))
    xpk_ref[...] = _pack_bf16_pairs(x)
    posin_s[pl.ds(i * RB, RB), :] = posin
    i1_s[pl.ds(i * RB, RB), :] = idx1
    i2_s[pl.ds(i * RB, RB), :] = idx2
    cnt_acc[...] += jnp.sum(mask, axis=0, keepdims=True)

    @pl.when(i == SB - 1)
    def _fin():
        counts = cnt_acc[...]
        nb = jnp.ceil(counts / T)
        lane8 = lax.broadcasted_iota(jnp.int32, (1, E), 1)
        baseb = jnp.zeros_like(nb)
        for e in range(E - 1):
            baseb += jnp.where(lane8 > e, nb[:, e:e + 1], 0.0)
        cum_incl = baseb + nb
        ntot = jnp.sum(nb, axis=1, keepdims=True)

        slotmat = baseb * T + posin_s[...]
        iota_se = lax.broadcasted_iota(jnp.int32, (S, E), 1)
        o1 = iota_se == i1_s[...]
        o2 = iota_se == i2_s[...]
        slot1_ref[...] = jnp.sum(jnp.where(o1, slotmat, 0.0),
                                 axis=1).astype(jnp.int32)
        slot2_ref[...] = jnp.sum(jnp.where(o2, slotmat, 0.0),
                                 axis=1).astype(jnp.int32)

        ib = lax.broadcasted_iota(jnp.int32, (1, NBLK), 1).astype(jnp.float32)
        raw = jnp.zeros((1, NBLK), jnp.float32)
        el = jnp.zeros((1, 1), jnp.float32)
        for e in range(E):
            raw += (cum_incl[:, e:e + 1] <= ib).astype(jnp.float32)
            el += (cum_incl[:, e:e + 1] <= ntot - 1.0).astype(jnp.float32)
        be_ref[...] = jnp.minimum(raw, el).astype(jnp.int32).reshape(NBLK)
        nt_ref[...] = ntot.astype(jnp.int32).reshape(1)


def _dispatch_body(xpk_hbm, s1_hbm, s2_hbm, xs_hbm, xb, i1, i2, sem):
    wid = lax.axis_index("s") * NC + lax.axis_index("c")
    base = wid * TOK_W
    l1 = pltpu.async_copy(s1_hbm.at[pl.ds(base, TOK_W)], i1, sem)
    l2 = pltpu.async_copy(s2_hbm.at[pl.ds(base, TOK_W)], i2, sem)
    l3 = pltpu.async_copy(xpk_hbm.at[pl.ds(base, TOK_W)], xb, sem)
    l1.wait()
    l2.wait()
    l3.wait()
    c1 = pltpu.async_copy(xb, xs_hbm.at[i1], sem)
    c2 = pltpu.async_copy(xb, xs_hbm.at[i2], sem)
    c1.wait()
    c2.wait()


def _mlp_body(be_ref, nt_ref, xs_ref, wi_ref, wv_ref, wo_ref, ys_ref):
    i = pl.program_id(0)
    j = pl.program_id(1)

    @pl.when(i < nt_ref[0])
    def _compute():
        xb = _unpack_bf16_pairs(xs_ref[...])
        a = jnp.dot(xb, wi_ref[0].astype(jnp.bfloat16),
                    preferred_element_type=jnp.float32)
        b = jnp.dot(xb, wv_ref[0].astype(jnp.bfloat16),
                    preferred_element_type=jnp.float32)
        h = (jax.nn.gelu(a) * b).astype(jnp.bfloat16)
        y = jnp.dot(h, wo_ref[0].astype(jnp.bfloat16),
                    preferred_element_type=jnp.float32)

        @pl.when(j == 0)
        def _set():
            ys_ref[...] = y

        @pl.when(j != 0)
        def _acc():
            ys_ref[...] += y


def _combine_body(ys_hbm, s1_hbm, s2_hbm, w1_hbm, w2_hbm, out_hbm,
                  i1, i2, wv1, wv2, b1, b2, sem):
    wid = lax.axis_index("s") * NC + lax.axis_index("c")
    base = wid * TOK_W
    l1 = pltpu.async_copy(s1_hbm.at[pl.ds(base, TOK_W)], i1, sem)
    l2 = pltpu.async_copy(s2_hbm.at[pl.ds(base, TOK_W)], i2, sem)
    l3 = pltpu.async_copy(w1_hbm.at[pl.ds(base, TOK_W)], wv1, sem)
    l4 = pltpu.async_copy(w2_hbm.at[pl.ds(base, TOK_W)], wv2, sem)
    l1.wait()
    l2.wait()
    l3.wait()
    l4.wait()
    c1 = pltpu.async_copy(ys_hbm.at[i1], b1, sem)
    c2 = pltpu.async_copy(ys_hbm.at[i2], b2, sem)
    c1.wait()
    c2.wait()

    def row(t, _):
        w1s = wv1[t]
        w2s = wv2[t]

        def col(c, _):
            b1[t, pl.ds(c * 16, 16)] = (b1[t, pl.ds(c * 16, 16)] * w1s
                                        + b2[t, pl.ds(c * 16, 16)] * w2s)
            return 0
        return lax.fori_loop(0, D // 16, col, 0, unroll=8)

    lax.fori_loop(0, TOK_W, row, 0)
    pltpu.sync_copy(b1, out_hbm.at[pl.ds(base, TOK_W)])


def kernel(hidden_states, gate_w, w_in, w_v, w_out):
    B = hidden_states.shape[0]
    x = hidden_states.reshape(S, D)

    logits, w1, w2, xpk, slot1, slot2, be, nt = pl.pallas_call(
        _router_body,
        grid=(SB,),
        in_specs=[
            pl.BlockSpec((RB, D), lambda i: (i, 0)),
            pl.BlockSpec((D, E), lambda i: (0, 0)),
        ],
        out_specs=[
            pl.BlockSpec((RB, E), lambda i: (i, 0)),
            pl.BlockSpec((RB, 16), lambda i: (i, 0)),
            pl.BlockSpec((RB, 16), lambda i: (i, 0)),
            pl.BlockSpec((RB, DP), lambda i: (i, 0)),
            pl.BlockSpec((S,), lambda i: (0,)),
            pl.BlockSpec((S,), lambda i: (0,)),
            pl.BlockSpec((NBLK,), lambda i: (0,)),
            pl.BlockSpec((1,), lambda i: (0,)),
        ],
        out_shape=(
            jax.ShapeDtypeStruct((S, E), jnp.float32),
            jax.ShapeDtypeStruct((S, 16), jnp.float32),
            jax.ShapeDtypeStruct((S, 16), jnp.float32),
            jax.ShapeDtypeStruct((S, DP), jnp.int32),
            jax.ShapeDtypeStruct((S,), jnp.int32),
            jax.ShapeDtypeStruct((S,), jnp.int32),
            jax.ShapeDtypeStruct((NBLK,), jnp.int32),
            jax.ShapeDtypeStruct((1,), jnp.int32),
        ),
        scratch_shapes=[
            pltpu.VMEM((1, E), jnp.float32),
            pltpu.VMEM((S, E), jnp.float32),
            pltpu.VMEM((S, 1), jnp.int32),
            pltpu.VMEM((S, 1), jnp.int32),
        ],
    )(x, gate_w)

    mesh = plsc.VectorSubcoreMesh(core_axis_name="c", subcore_axis_name="s")
    xs = pl.kernel(
        _dispatch_body,
        out_type=jax.ShapeDtypeStruct((NSLOT, DP), jnp.int32),
        mesh=mesh,
        scratch_types=[
            pltpu.VMEM((TOK_W, DP), jnp.int32),
            pltpu.VMEM((TOK_W,), jnp.int32),
            pltpu.VMEM((TOK_W,), jnp.int32),
            pltpu.SemaphoreType.DMA,
        ],
    )(xpk, slot1, slot2)

    ys = pl.pallas_call(
        _mlp_body,
        grid_spec=pltpu.PrefetchScalarGridSpec(
            num_scalar_prefetch=2,
            grid=(NBLK, JF),
            in_specs=[
                pl.BlockSpec(
                    (T, DP), lambda i, j, be, nt: (jnp.minimum(i, nt[0] - 1), 0)),
                pl.BlockSpec(
                    (1, D, Fb),
                    lambda i, j, be, nt: (be[i], 0,
                                          jnp.where(i < nt[0], j, JF - 1))),
                pl.BlockSpec(
                    (1, D, Fb),
                    lambda i, j, be, nt: (be[i], 0,
                                          jnp.where(i < nt[0], j, JF - 1))),
                pl.BlockSpec(
                    (1, Fb, D),
                    lambda i, j, be, nt: (be[i],
                                          jnp.where(i < nt[0], j, JF - 1), 0)),
            ],
            out_specs=pl.BlockSpec(
                (T, D), lambda i, j, be, nt: (jnp.minimum(i, nt[0] - 1), 0)),
        ),
        out_shape=jax.ShapeDtypeStruct((NSLOT, D), jnp.float32),
    )(be, nt, xs, w_in, w_v, w_out)

    out = pl.kernel(
        _combine_body,
        out_type=jax.ShapeDtypeStruct((S, D), jnp.float32),
        mesh=mesh,
        scratch_types=[
            pltpu.VMEM((TOK_W,), jnp.int32),
            pltpu.VMEM((TOK_W,), jnp.int32),
            pltpu.VMEM((TOK_W, 16), jnp.float32),
            pltpu.VMEM((TOK_W, 16), jnp.float32),
            pltpu.VMEM((TOK_W, D), jnp.float32),
            pltpu.VMEM((TOK_W, D), jnp.float32),
            pltpu.SemaphoreType.DMA,
        ],
    )(ys, slot1, slot2, w1, w2)

    return out.reshape(B, S, D), logits.reshape(B, S, E)
